# Initial kernel scaffold; baseline (speedup 1.0000x reference)
import jax
import jax.numpy as jnp
from jax.experimental import pallas as pl


def kernel(x, W_enc, b_enc, dec_bias):
    raise NotImplementedError("write your pallas kernel here")


# speedup vs baseline: 16.2540x; 16.2540x over previous
import functools

import jax
import jax.numpy as jnp
from jax.experimental import pallas as pl
from jax.experimental.pallas import tpu as pltpu

INPUT_DIM = 768
HIDDEN_DIM = 8192
TOPK = 64
BLOCK_ROWS = 128


def _fused_kernel(x_ref, w_ref, b_ref, dbias_ref, sparse_ref, recon_ref, part_ref):
    x = x_ref[...]
    w = w_ref[...]
    h = jax.lax.dot_general(
        x, w, (((1,), (1,)), ((), ())),
        preferred_element_type=jnp.float32,
    ) + b_ref[...]

    bits = jax.lax.bitcast_convert_type(jnp.abs(h), jnp.int32)

    def body(i, thr):
        cand = thr | jnp.left_shift(jnp.int32(1), 30 - i)
        cnt = jnp.sum((bits >= cand).astype(jnp.int32), axis=1, keepdims=True)
        return jnp.where(cnt >= TOPK, cand, thr)

    thr = jax.lax.fori_loop(
        0, 31, body, jnp.zeros((x.shape[0], 1), jnp.int32))

    s = jnp.where(bits >= thr, h, 0.0)
    sparse_ref[...] = s

    r = jax.lax.dot_general(
        s, w, (((1,), (0,)), ((), ())),
        preferred_element_type=jnp.float32,
    ) + dbias_ref[...]
    recon_ref[...] = r

    d = r - x
    lane = jax.lax.broadcasted_iota(jnp.int32, (1, 1, 128), 2)
    part = (
        jnp.where(lane == 0, jnp.sum(d * d), 0.0)
        + jnp.where(lane == 1, jnp.sum((s != 0.0).astype(jnp.float32)), 0.0)
        + jnp.where(lane == 2, jnp.sum(jnp.abs(s)), 0.0)
        + jnp.where(lane == 3, jnp.sum(s), 0.0)
        + jnp.where(lane == 4, jnp.max(s), 0.0)
    )
    part_ref[...] = part


@jax.jit
def kernel(x, W_enc, b_enc, dec_bias):
    B = x.shape[0]
    nb = B // BLOCK_ROWS
    sparse, recon, part = pl.pallas_call(
        _fused_kernel,
        grid=(nb,),
        in_specs=[
            pl.BlockSpec((BLOCK_ROWS, INPUT_DIM), lambda r: (r, 0)),
            pl.BlockSpec((HIDDEN_DIM, INPUT_DIM), lambda r: (0, 0)),
            pl.BlockSpec((1, HIDDEN_DIM), lambda r: (0, 0)),
            pl.BlockSpec((1, INPUT_DIM), lambda r: (0, 0)),
        ],
        out_specs=[
            pl.BlockSpec((BLOCK_ROWS, HIDDEN_DIM), lambda r: (r, 0)),
            pl.BlockSpec((BLOCK_ROWS, INPUT_DIM), lambda r: (r, 0)),
            pl.BlockSpec((1, 1, 128), lambda r: (r, 0, 0)),
        ],
        out_shape=[
            jax.ShapeDtypeStruct((B, HIDDEN_DIM), jnp.float32),
            jax.ShapeDtypeStruct((B, INPUT_DIM), jnp.float32),
            jax.ShapeDtypeStruct((nb, 1, 128), jnp.float32),
        ],
        compiler_params=pltpu.CompilerParams(
            dimension_semantics=("parallel",),
        ),
    )(x, W_enc, b_enc.reshape(1, HIDDEN_DIM), dec_bias.reshape(1, INPUT_DIM))

    recon_loss = part[:, 0, 0].sum() / (B * INPUT_DIM)
    num_active = part[:, 0, 1].sum() / B
    sparsity_ratio = num_active / HIDDEN_DIM
    l1_loss = part[:, 0, 2].sum() / (B * HIDDEN_DIM)
    mean_activation = part[:, 0, 3].sum() / (B * HIDDEN_DIM)
    max_activation = part[:, 0, 4].max()
    return (recon, sparse, recon_loss, l1_loss, num_active, sparsity_ratio,
            mean_activation, max_activation)
